# baseline (speedup 1.0000x reference)
import functools

import jax
import jax.numpy as jnp
from jax import lax
from jax.experimental import pallas as pl
from jax.experimental.pallas import tpu as pltpu
from jax.experimental.pallas import tpu_sc as plsc

N = 10000
E = 320000
DIN = 128
DM = 256
DE = 16

NC = 2
NS = 16
NW = NC * NS

NT = 10240
ROWS_PER_TILE = NT // NS
DUMMY = N

EPAD = 327680
EW5 = EPAD // NW
EW3 = EPAD // NS
CH = 128
CH5 = 80
K5_0 = (EPAD // CH5 // NS) // 2
K5_1 = (EPAD // CH5 // NS) - K5_0

_mesh = lambda: plsc.VectorSubcoreMesh(
    core_axis_name="c", subcore_axis_name="s", num_cores=NC, num_subcores=NS)


def _deg_body(dst_hbm, ones_hbm, zeros_hbm, out_hbm, sh_deg, idx_v, ones_v,
              si0, si1):
    c = lax.axis_index("c")
    s = lax.axis_index("s")
    w = c * NS + s
    nch = (EPAD // NW) // CH
    sems = (si0, si1)
    pltpu.sync_copy(ones_hbm, ones_v)
    r0 = s * ROWS_PER_TILE
    pltpu.sync_copy(zeros_hbm.at[pl.ds(r0, ROWS_PER_TILE)],
                    sh_deg.at[pl.ds(r0, ROWS_PER_TILE)])
    plsc.subcore_barrier()

    def issue(ch, t):
        base = w * (EPAD // NW) + ch * CH
        pltpu.async_copy(dst_hbm.at[pl.ds(base, CH)], idx_v.at[t], sems[t])

    issue(0, 0)

    def outer(g, _):
        for t in (0, 1):
            ch = g * 2 + t
            pltpu.make_async_copy(dst_hbm.at[pl.ds(0, CH)], idx_v.at[t],
                                  sems[t]).wait()
            nxt = ch + 1
            pl.when(nxt < nch)(lambda: issue(nxt, 1 - t))
            pltpu.sync_copy(ones_v, sh_deg.at[idx_v.at[t]], add=True)
        return _

    lax.fori_loop(0, nch // 2, outer, None)
    plsc.subcore_barrier()
    pltpu.sync_copy(sh_deg.at[pl.ds(r0, ROWS_PER_TILE)],
                    out_hbm.at[c, pl.ds(r0, ROWS_PER_TILE)])


def _segsum_body(y2_hbm, src_hbm, dst_hbm, zeros_hbm, out_hbm,
                 sh_s, sidx_v, didx_v, rows_v, sem0, sem1):
    c = lax.axis_index("c")
    s = lax.axis_index("s")
    r0 = s * ROWS_PER_TILE
    pltpu.sync_copy(zeros_hbm.at[pl.ds(r0, ROWS_PER_TILE)],
                    sh_s.at[pl.ds(r0, ROWS_PER_TILE)])
    plsc.subcore_barrier()

    def run(y_ref):
        nch = EW3 // CH
        sems = (sem0, sem1)

        def issue(ch, t):
            base = s * EW3 + ch * CH
            pltpu.sync_copy(src_hbm.at[pl.ds(base, CH)], sidx_v.at[t])
            pltpu.sync_copy(dst_hbm.at[pl.ds(base, CH)], didx_v.at[t])
            pltpu.async_copy(y_ref.at[sidx_v.at[t]], rows_v.at[t], sems[t])

        issue(0, 0)

        def outer(g, _):
            for t in (0, 1):
                ch = g * 2 + t
                pltpu.make_async_copy(y_ref.at[sidx_v.at[t]], rows_v.at[t],
                                      sems[t]).wait()
                nxt = ch + 1
                pl.when(nxt < nch)(lambda: issue(nxt, 1 - t))
                pltpu.sync_copy(rows_v.at[t], sh_s.at[didx_v.at[t]], add=True)
            return _

        lax.fori_loop(0, nch // 2, outer, None)

    pl.when(c == 0)(lambda: run(y2_hbm.at[0]))
    pl.when(c == 1)(lambda: run(y2_hbm.at[1]))
    plsc.subcore_barrier()
    pltpu.sync_copy(sh_s.at[pl.ds(r0, ROWS_PER_TILE)],
                    out_hbm.at[c, pl.ds(r0, ROWS_PER_TILE)])


def _edge_gather_body(a_hbm, b_hbm, src_hbm, dst_hbm, za_hbm, zb_hbm,
                      sidx_v, didx_v, ra_v, rb_v,
                      sa0, sa1, sb0, sb1):
    c = lax.axis_index("c")
    s = lax.axis_index("s")
    sems_a = (sa0, sa1)
    sems_b = (sb0, sb1)
    k_tile = src_hbm.shape[0] // (CH5 * NW)
    nch = k_tile
    cbase = (c * NS + s) * k_tile

    def issue(ch, t):
        base = (cbase + ch) * CH5
        pltpu.sync_copy(src_hbm.at[pl.ds(base, CH5)], sidx_v.at[t])
        pltpu.sync_copy(dst_hbm.at[pl.ds(base, CH5)], didx_v.at[t])
        pltpu.async_copy(a_hbm.at[sidx_v.at[t]], ra_v.at[t], sems_a[t])
        pltpu.async_copy(b_hbm.at[didx_v.at[t]], rb_v.at[t], sems_b[t])

    issue(0, 0)

    def outer(g, _):
        for t in (0, 1):
            ch = g * 2 + t
            nxt = ch + 1
            pl.when(nxt < nch)(lambda: issue(nxt, 1 - t))
            pltpu.make_async_copy(a_hbm.at[sidx_v.at[t]], ra_v.at[t],
                                  sems_a[t]).wait()
            pltpu.make_async_copy(b_hbm.at[didx_v.at[t]], rb_v.at[t],
                                  sems_b[t]).wait()
            base = (cbase + ch) * CH5
            pltpu.sync_copy(ra_v.at[t], za_hbm.at[pl.ds(base, CH5)])
            pltpu.sync_copy(rb_v.at[t], zb_hbm.at[pl.ds(base, CH5)])
        return _

    lax.fori_loop(0, nch // 2, outer, None)


def _build_sc_kernels(interpret=False):
    deg_k = functools.partial(
        pl.kernel,
        out_type=jax.ShapeDtypeStruct((NC, NT, DIN), jnp.float32),
        mesh=_mesh(),
        scratch_types=[
            pltpu.VMEM_SHARED((NT, DIN), jnp.float32),
            pltpu.VMEM((2, CH), jnp.int32),
            pltpu.VMEM((CH, DIN), jnp.float32),
            pltpu.SemaphoreType.DMA,
            pltpu.SemaphoreType.DMA,
        ],
        interpret=interpret,
    )(_deg_body)
    segsum_k = functools.partial(
        pl.kernel,
        out_type=jax.ShapeDtypeStruct((NC, NT, DIN), jnp.float32),
        mesh=_mesh(),
        scratch_types=[
            pltpu.VMEM_SHARED((NT, DIN), jnp.float32),
            pltpu.VMEM((2, CH), jnp.int32),
            pltpu.VMEM((2, CH), jnp.int32),
            pltpu.VMEM((2, CH, DIN), jnp.float32),
            pltpu.SemaphoreType.DMA,
            pltpu.SemaphoreType.DMA,
        ],
        interpret=interpret,
    )(_segsum_body)
    edge_gather_k = functools.partial(
        pl.kernel,
        out_type=[
            jax.ShapeDtypeStruct((EPAD // 2, DM), jnp.int32),
            jax.ShapeDtypeStruct((EPAD // 2, DM), jnp.int32),
        ],
        mesh=_mesh(),
        scratch_types=[
            pltpu.VMEM((2, CH5), jnp.int32),
            pltpu.VMEM((2, CH5), jnp.int32),
            pltpu.VMEM((2, CH5, DM), jnp.int32),
            pltpu.VMEM((2, CH5, DM), jnp.int32),
            pltpu.SemaphoreType.DMA,
            pltpu.SemaphoreType.DMA,
            pltpu.SemaphoreType.DMA,
            pltpu.SemaphoreType.DMA,
        ],
        interpret=interpret,
    )(_edge_gather_body)
    return deg_k, segsum_k, edge_gather_k


_deg_kernel, _segsum_kernel, _edge_gather_kernel = _build_sc_kernels()


def _xw_body(x_ref, w_ref, degs_ref, y2_ref):
    deg = degs_ref[0, :, 0] + degs_ref[1, :, 0] + 1.0
    dinv = lax.rsqrt(deg)
    xw = jnp.dot(x_ref[...], w_ref[...], preferred_element_type=jnp.float32)
    y = xw * dinv[:, None]
    y2_ref[0] = y[:, :DIN]
    y2_ref[1] = y[:, DIN:]


def _emb_body(s2_ref, y2_ref, degs_ref, w1_ref, b_ref, b1_ref, a_ref, bt_ref):
    deg = degs_ref[0, :, 0] + degs_ref[1, :, 0] + 1.0
    dinv = lax.rsqrt(deg)
    sy = jnp.concatenate([s2_ref[0] + y2_ref[0], s2_ref[1] + y2_ref[1]], axis=1)
    emb = sy * dinv[:, None] + b_ref[...][None, :]
    a = (jnp.dot(emb, w1_ref[:DM, :], preferred_element_type=jnp.float32)
         + b1_ref[...][None, :])
    bt = jnp.dot(emb, w1_ref[DM:, :], preferred_element_type=jnp.float32)

    def pack(v):
        lo = jax.lax.bitcast_convert_type(
            v[:, :DM].astype(jnp.bfloat16), jnp.uint16).astype(jnp.int32)
        hi = jax.lax.bitcast_convert_type(
            v[:, DM:].astype(jnp.bfloat16), jnp.uint16).astype(jnp.int32)
        return lo | (hi << 16)

    a_ref[...] = pack(a)
    bt_ref[...] = pack(bt)


def _out_body(za_ref, zb_ref, w2_ref, b2_ref, o_ref):
    za = za_ref[...]
    zb = zb_ref[...]
    himask = jnp.int32(-65536)
    _f = lambda v: jax.lax.bitcast_convert_type(v, jnp.float32)
    h_lo = jax.nn.relu(_f(za << 16) + _f(zb << 16))
    h_hi = jax.nn.relu(_f(za & himask) + _f(zb & himask))
    o_ref[...] = (jnp.dot(h_lo, w2_ref[:DM, :],
                          preferred_element_type=jnp.float32)
                  + jnp.dot(h_hi, w2_ref[DM:, :],
                            preferred_element_type=jnp.float32)
                  + b2_ref[...][None, :])


def kernel(x, edge_index, W, b, W1, b1, W2, b2):
    src = edge_index[0].astype(jnp.int32)
    dst = edge_index[1].astype(jnp.int32)
    padn = EPAD - E
    zpad = jnp.zeros((padn,), jnp.int32)
    src_p = jnp.concatenate([src, zpad])
    dst_g = jnp.concatenate([dst, zpad])
    dst_s = jnp.concatenate([dst, jnp.full((padn,), DUMMY, jnp.int32)])
    ones_c = jnp.ones((CH, DIN), jnp.float32)
    zeros_s = jnp.zeros((NT, DIN), jnp.float32)

    degs = _deg_kernel(dst_s, ones_c, zeros_s)

    rb = 1000
    grid_n = N // rb
    y2 = pl.pallas_call(
        _xw_body,
        grid=(grid_n,),
        in_specs=[
            pl.BlockSpec((rb, DIN), lambda i: (i, 0)),
            pl.BlockSpec((DIN, DM), lambda i: (0, 0)),
            pl.BlockSpec((NC, rb, DIN), lambda i: (0, i, 0)),
        ],
        out_specs=pl.BlockSpec((NC, rb, DIN), lambda i: (0, i, 0)),
        out_shape=jax.ShapeDtypeStruct((NC, N, DIN), jnp.float32),
    )(x, W, degs)

    s2 = _segsum_kernel(y2, src_p, dst_s, zeros_s)

    a_tab, bt_tab = pl.pallas_call(
        _emb_body,
        grid=(grid_n,),
        in_specs=[
            pl.BlockSpec((NC, rb, DIN), lambda i: (0, i, 0)),
            pl.BlockSpec((NC, rb, DIN), lambda i: (0, i, 0)),
            pl.BlockSpec((NC, rb, DIN), lambda i: (0, i, 0)),
            pl.BlockSpec((2 * DM, 2 * DM), lambda i: (0, 0)),
            pl.BlockSpec((DM,), lambda i: (0,)),
            pl.BlockSpec((2 * DM,), lambda i: (0,)),
        ],
        out_specs=[
            pl.BlockSpec((rb, DM), lambda i: (i, 0)),
            pl.BlockSpec((rb, DM), lambda i: (i, 0)),
        ],
        out_shape=[
            jax.ShapeDtypeStruct((N, DM), jnp.int32),
            jax.ShapeDtypeStruct((N, DM), jnp.int32),
        ],
    )(s2, y2, degs, W1, b, b1)

    eb = 1024
    half = EPAD // 2

    def out_half(za_i32, zb_i32, n_rows):
        return pl.pallas_call(
            _out_body,
            grid=(n_rows // eb,),
            in_specs=[
                pl.BlockSpec((eb, DM), lambda i: (i, 0)),
                pl.BlockSpec((eb, DM), lambda i: (i, 0)),
                pl.BlockSpec((2 * DM, DE), lambda i: (0, 0)),
                pl.BlockSpec((DE,), lambda i: (0,)),
            ],
            out_specs=pl.BlockSpec((eb, DE), lambda i: (i, 0)),
            out_shape=jax.ShapeDtypeStruct((n_rows, DE), jnp.float32),
        )(za_i32, zb_i32, W2, b2)

    za0, zb0 = _edge_gather_kernel(a_tab, bt_tab, src_p[:half], dst_g[:half])
    za1, zb1 = _edge_gather_kernel(a_tab, bt_tab, src_p[half:], dst_g[half:])
    out0 = out_half(za0, zb0, half)
    out1 = out_half(za1, zb1, half)
    return jnp.concatenate([out0, out1], axis=0)[:E]


# speedup vs baseline: 1.0228x; 1.0228x over previous
import functools

import jax
import jax.numpy as jnp
from jax import lax
from jax.experimental import pallas as pl
from jax.experimental.pallas import tpu as pltpu
from jax.experimental.pallas import tpu_sc as plsc

N = 10000
E = 320000
DIN = 128
DM = 256
DE = 16

NC = 2
NS = 16
NW = NC * NS

NT = 10240
ROWS_PER_TILE = NT // NS
DUMMY = N

EPAD = 327680
EW5 = EPAD // NW
EW3 = EPAD // NS
CH = 128
CH5 = 80
K5_0 = (EPAD // CH5 // NS) // 2
K5_1 = (EPAD // CH5 // NS) - K5_0

_mesh = lambda: plsc.VectorSubcoreMesh(
    core_axis_name="c", subcore_axis_name="s", num_cores=NC, num_subcores=NS)


def _deg_body(dst_hbm, ones_hbm, zeros_hbm, out_hbm, sh_deg, idx_v, ones_v,
              si0, si1):
    c = lax.axis_index("c")
    s = lax.axis_index("s")
    w = c * NS + s
    nch = (EPAD // NW) // CH
    sems = (si0, si1)
    pltpu.sync_copy(ones_hbm, ones_v)
    r0 = s * ROWS_PER_TILE
    pltpu.sync_copy(zeros_hbm.at[pl.ds(r0, ROWS_PER_TILE)],
                    sh_deg.at[pl.ds(r0, ROWS_PER_TILE)])
    plsc.subcore_barrier()

    def issue(ch, t):
        base = w * (EPAD // NW) + ch * CH
        pltpu.async_copy(dst_hbm.at[pl.ds(base, CH)], idx_v.at[t], sems[t])

    issue(0, 0)

    def outer(g, _):
        for t in (0, 1):
            ch = g * 2 + t
            pltpu.make_async_copy(dst_hbm.at[pl.ds(0, CH)], idx_v.at[t],
                                  sems[t]).wait()
            nxt = ch + 1
            pl.when(nxt < nch)(lambda: issue(nxt, 1 - t))
            pltpu.sync_copy(ones_v, sh_deg.at[idx_v.at[t]], add=True)
        return _

    lax.fori_loop(0, nch // 2, outer, None)
    plsc.subcore_barrier()
    pltpu.sync_copy(sh_deg.at[pl.ds(r0, ROWS_PER_TILE)],
                    out_hbm.at[c, pl.ds(r0, ROWS_PER_TILE)])


def _segsum_body(y2_hbm, src_hbm, dst_hbm, zeros_hbm, out_hbm,
                 sh_s, sidx_v, didx_v, rows_v, sem0, sem1):
    c = lax.axis_index("c")
    s = lax.axis_index("s")
    r0 = s * ROWS_PER_TILE
    pltpu.sync_copy(zeros_hbm.at[pl.ds(r0, ROWS_PER_TILE)],
                    sh_s.at[pl.ds(r0, ROWS_PER_TILE)])
    plsc.subcore_barrier()

    def run(y_ref):
        nch = EW3 // CH
        sems = (sem0, sem1)

        def issue(ch, t):
            base = s * EW3 + ch * CH
            pltpu.sync_copy(src_hbm.at[pl.ds(base, CH)], sidx_v.at[t])
            pltpu.sync_copy(dst_hbm.at[pl.ds(base, CH)], didx_v.at[t])
            pltpu.async_copy(y_ref.at[sidx_v.at[t]], rows_v.at[t], sems[t])

        issue(0, 0)

        def outer(g, _):
            for t in (0, 1):
                ch = g * 2 + t
                pltpu.make_async_copy(y_ref.at[sidx_v.at[t]], rows_v.at[t],
                                      sems[t]).wait()
                nxt = ch + 1
                pl.when(nxt < nch)(lambda: issue(nxt, 1 - t))
                pltpu.sync_copy(rows_v.at[t], sh_s.at[didx_v.at[t]], add=True)
            return _

        lax.fori_loop(0, nch // 2, outer, None)

    pl.when(c == 0)(lambda: run(y2_hbm.at[0]))
    pl.when(c == 1)(lambda: run(y2_hbm.at[1]))
    plsc.subcore_barrier()
    pltpu.sync_copy(sh_s.at[pl.ds(r0, ROWS_PER_TILE)],
                    out_hbm.at[c, pl.ds(r0, ROWS_PER_TILE)])


def _edge_gather_body(a_hbm, b_hbm, src_hbm, dst_hbm, za_hbm, zb_hbm,
                      sidx_v, didx_v, ra_v, rb_v,
                      sa0, sa1, sb0, sb1):
    c = lax.axis_index("c")
    s = lax.axis_index("s")
    sems_a = (sa0, sa1)
    sems_b = (sb0, sb1)
    nch = jnp.where(c == 0, K5_0, K5_1)
    cbase = jnp.where(c == 0, s * K5_0, NS * K5_0 + s * K5_1)

    def issue(ch, t):
        base = (cbase + ch) * CH5
        pltpu.sync_copy(src_hbm.at[pl.ds(base, CH5)], sidx_v.at[t])
        pltpu.sync_copy(dst_hbm.at[pl.ds(base, CH5)], didx_v.at[t])
        pltpu.async_copy(a_hbm.at[sidx_v.at[t]], ra_v.at[t], sems_a[t])
        pltpu.async_copy(b_hbm.at[didx_v.at[t]], rb_v.at[t], sems_b[t])

    issue(0, 0)

    def outer(g, _):
        for t in (0, 1):
            ch = g * 2 + t
            nxt = ch + 1
            pl.when(nxt < nch)(lambda: issue(nxt, 1 - t))
            pltpu.make_async_copy(a_hbm.at[sidx_v.at[t]], ra_v.at[t],
                                  sems_a[t]).wait()
            pltpu.make_async_copy(b_hbm.at[didx_v.at[t]], rb_v.at[t],
                                  sems_b[t]).wait()
            base = (cbase + ch) * CH5
            pltpu.sync_copy(ra_v.at[t], za_hbm.at[pl.ds(base, CH5)])
            pltpu.sync_copy(rb_v.at[t], zb_hbm.at[pl.ds(base, CH5)])
        return _

    lax.fori_loop(0, nch // 2, outer, None)


def _build_sc_kernels(interpret=False):
    deg_k = functools.partial(
        pl.kernel,
        out_type=jax.ShapeDtypeStruct((NC, NT, DIN), jnp.float32),
        mesh=_mesh(),
        scratch_types=[
            pltpu.VMEM_SHARED((NT, DIN), jnp.float32),
            pltpu.VMEM((2, CH), jnp.int32),
            pltpu.VMEM((CH, DIN), jnp.float32),
            pltpu.SemaphoreType.DMA,
            pltpu.SemaphoreType.DMA,
        ],
        interpret=interpret,
    )(_deg_body)
    segsum_k = functools.partial(
        pl.kernel,
        out_type=jax.ShapeDtypeStruct((NC, NT, DIN), jnp.float32),
        mesh=_mesh(),
        scratch_types=[
            pltpu.VMEM_SHARED((NT, DIN), jnp.float32),
            pltpu.VMEM((2, CH), jnp.int32),
            pltpu.VMEM((2, CH), jnp.int32),
            pltpu.VMEM((2, CH, DIN), jnp.float32),
            pltpu.SemaphoreType.DMA,
            pltpu.SemaphoreType.DMA,
        ],
        interpret=interpret,
    )(_segsum_body)
    edge_gather_k = functools.partial(
        pl.kernel,
        out_type=[
            jax.ShapeDtypeStruct((EPAD, DM), jnp.int32),
            jax.ShapeDtypeStruct((EPAD, DM), jnp.int32),
        ],
        mesh=_mesh(),
        scratch_types=[
            pltpu.VMEM((2, CH5), jnp.int32),
            pltpu.VMEM((2, CH5), jnp.int32),
            pltpu.VMEM((2, CH5, DM), jnp.int32),
            pltpu.VMEM((2, CH5, DM), jnp.int32),
            pltpu.SemaphoreType.DMA,
            pltpu.SemaphoreType.DMA,
            pltpu.SemaphoreType.DMA,
            pltpu.SemaphoreType.DMA,
        ],
        interpret=interpret,
    )(_edge_gather_body)
    return deg_k, segsum_k, edge_gather_k


_deg_kernel, _segsum_kernel, _edge_gather_kernel = _build_sc_kernels()


def _xw_body(x_ref, w_ref, degs_ref, y2_ref):
    deg = degs_ref[0, :, 0] + degs_ref[1, :, 0] + 1.0
    dinv = lax.rsqrt(deg)
    xw = jnp.dot(x_ref[...], w_ref[...], preferred_element_type=jnp.float32)
    y = xw * dinv[:, None]
    y2_ref[0] = y[:, :DIN]
    y2_ref[1] = y[:, DIN:]


def _emb_body(s2_ref, y2_ref, degs_ref, w1_ref, b_ref, b1_ref, a_ref, bt_ref):
    deg = degs_ref[0, :, 0] + degs_ref[1, :, 0] + 1.0
    dinv = lax.rsqrt(deg)
    sy = jnp.concatenate([s2_ref[0] + y2_ref[0], s2_ref[1] + y2_ref[1]], axis=1)
    emb = sy * dinv[:, None] + b_ref[...][None, :]
    a = (jnp.dot(emb, w1_ref[:DM, :], preferred_element_type=jnp.float32)
         + b1_ref[...][None, :])
    bt = jnp.dot(emb, w1_ref[DM:, :], preferred_element_type=jnp.float32)

    def pack(v):
        lo = jax.lax.bitcast_convert_type(
            v[:, :DM].astype(jnp.bfloat16), jnp.uint16).astype(jnp.int32)
        hi = jax.lax.bitcast_convert_type(
            v[:, DM:].astype(jnp.bfloat16), jnp.uint16).astype(jnp.int32)
        return lo | (hi << 16)

    a_ref[...] = pack(a)
    bt_ref[...] = pack(bt)


def _out_body(za_ref, zb_ref, w2_ref, b2_ref, o_ref):
    za = za_ref[...]
    zb = zb_ref[...]
    himask = jnp.int32(-65536)
    _f = lambda v: jax.lax.bitcast_convert_type(v, jnp.float32)
    h_lo = jax.nn.relu(_f(za << 16) + _f(zb << 16))
    h_hi = jax.nn.relu(_f(za & himask) + _f(zb & himask))
    o_ref[...] = (jnp.dot(h_lo, w2_ref[:DM, :],
                          preferred_element_type=jnp.float32)
                  + jnp.dot(h_hi, w2_ref[DM:, :],
                            preferred_element_type=jnp.float32)
                  + b2_ref[...][None, :])


def kernel(x, edge_index, W, b, W1, b1, W2, b2):
    src = edge_index[0].astype(jnp.int32)
    dst = edge_index[1].astype(jnp.int32)
    padn = EPAD - E
    zpad = jnp.zeros((padn,), jnp.int32)
    src_p = jnp.concatenate([src, zpad])
    dst_g = jnp.concatenate([dst, zpad])
    dst_s = jnp.concatenate([dst, jnp.full((padn,), DUMMY, jnp.int32)])
    ones_c = jnp.ones((CH, DIN), jnp.float32)
    zeros_s = jnp.zeros((NT, DIN), jnp.float32)

    degs = _deg_kernel(dst_s, ones_c, zeros_s)

    rb = 1000
    grid_n = N // rb
    y2 = pl.pallas_call(
        _xw_body,
        grid=(grid_n,),
        in_specs=[
            pl.BlockSpec((rb, DIN), lambda i: (i, 0)),
            pl.BlockSpec((DIN, DM), lambda i: (0, 0)),
            pl.BlockSpec((NC, rb, DIN), lambda i: (0, i, 0)),
        ],
        out_specs=pl.BlockSpec((NC, rb, DIN), lambda i: (0, i, 0)),
        out_shape=jax.ShapeDtypeStruct((NC, N, DIN), jnp.float32),
    )(x, W, degs)

    s2 = _segsum_kernel(y2, src_p, dst_s, zeros_s)

    a_tab, bt_tab = pl.pallas_call(
        _emb_body,
        grid=(grid_n,),
        in_specs=[
            pl.BlockSpec((NC, rb, DIN), lambda i: (0, i, 0)),
            pl.BlockSpec((NC, rb, DIN), lambda i: (0, i, 0)),
            pl.BlockSpec((NC, rb, DIN), lambda i: (0, i, 0)),
            pl.BlockSpec((2 * DM, 2 * DM), lambda i: (0, 0)),
            pl.BlockSpec((DM,), lambda i: (0,)),
            pl.BlockSpec((2 * DM,), lambda i: (0,)),
        ],
        out_specs=[
            pl.BlockSpec((rb, DM), lambda i: (i, 0)),
            pl.BlockSpec((rb, DM), lambda i: (i, 0)),
        ],
        out_shape=[
            jax.ShapeDtypeStruct((N, DM), jnp.int32),
            jax.ShapeDtypeStruct((N, DM), jnp.int32),
        ],
    )(s2, y2, degs, W1, b, b1)

    za_i32, zb_i32 = _edge_gather_kernel(a_tab, bt_tab, src_p, dst_g)

    eb = 1000
    out = pl.pallas_call(
        _out_body,
        grid=(E // eb,),
        in_specs=[
            pl.BlockSpec((eb, DM), lambda i: (i, 0)),
            pl.BlockSpec((eb, DM), lambda i: (i, 0)),
            pl.BlockSpec((2 * DM, DE), lambda i: (0, 0)),
            pl.BlockSpec((DE,), lambda i: (0,)),
        ],
        out_specs=pl.BlockSpec((eb, DE), lambda i: (i, 0)),
        out_shape=jax.ShapeDtypeStruct((E, DE), jnp.float32),
    )(za_i32, zb_i32, W2, b2)
    return out
